# baseline (speedup 1.0000x reference)
import functools

import jax
import jax.numpy as jnp
from jax import lax
from jax.experimental import pallas as pl
from jax.experimental.pallas import tpu as pltpu
from jax.experimental.pallas import tpu_sc as plsc

BATCH = 64
CHANNELS = 192
H = 32
W = 32
HW = H * W
FLAT = CHANNELS * HW
LANES = 128
ROWS_PER_BATCH = FLAT // LANES

_NUM_WORKERS = 32
_C_PER_W = CHANNELS // _NUM_WORKERS
_L = 16


def _sc_build_table(emb0_hbm, emb1_hbm, e_hbm, e0_v, e1_v, e_v):
    core = lax.axis_index("c")
    sub = lax.axis_index("s")
    wid = sub * 2 + core
    c0 = wid * _C_PER_W

    pltpu.sync_copy(emb0_hbm, e0_v)
    pltpu.sync_copy(emb1_hbm, e1_v)

    iota = lax.iota(jnp.int32, _L)
    for j in range(_C_PER_W):
        c = c0 + j
        cvec = jnp.full((_L,), 1, jnp.int32) * c
        col1 = [
            plsc.load_gather(e1_v, [(iota + _L * k) * CHANNELS + cvec])
            for k in range(W // _L)
        ]
        for h in range(H):
            b0 = plsc.load_gather(
                e0_v, [jnp.full((_L,), h * CHANNELS, jnp.int32) + cvec]
            )
            for k in range(W // _L):
                e_v[pl.ds(j * HW + h * W + k * _L, _L)] = b0 + col1[k]

    pltpu.sync_copy(e_v, e_hbm.at[pl.ds(c0 * HW, _C_PER_W * HW)])


@functools.partial(
    pl.kernel,
    out_type=jax.ShapeDtypeStruct((FLAT,), jnp.float32),
    mesh=plsc.VectorSubcoreMesh(core_axis_name="c", subcore_axis_name="s"),
    compiler_params=pltpu.CompilerParams(needs_layout_passes=False),
    scratch_types=[
        pltpu.VMEM((H * CHANNELS,), jnp.float32),
        pltpu.VMEM((W * CHANNELS,), jnp.float32),
        pltpu.VMEM((_C_PER_W * HW,), jnp.float32),
    ],
)
def _sc_table(emb0_hbm, emb1_hbm, e_hbm, e0_v, e1_v, e_v):
    _sc_build_table(emb0_hbm, emb1_hbm, e_hbm, e0_v, e1_v, e_v)


_CB = 8
_BLK_ROWS = ROWS_PER_BATCH * _CB


def _tc_add_body(x_ref, e_ref, o_ref):
    xv = x_ref[...].reshape(_CB, ROWS_PER_BATCH, LANES)
    o_ref[...] = (xv + e_ref[...][None]).reshape(_BLK_ROWS, LANES)


def _tc_add(x2, e2):
    return pl.pallas_call(
        _tc_add_body,
        grid=(BATCH // _CB,),
        in_specs=[
            pl.BlockSpec((_BLK_ROWS, LANES), lambda i: (i, 0)),
            pl.BlockSpec((ROWS_PER_BATCH, LANES), lambda i: (0, 0)),
        ],
        out_specs=pl.BlockSpec((_BLK_ROWS, LANES), lambda i: (i, 0)),
        out_shape=jax.ShapeDtypeStruct(
            (BATCH * ROWS_PER_BATCH, LANES), jnp.float32
        ),
        compiler_params=pltpu.CompilerParams(
            dimension_semantics=("arbitrary",),
        ),
    )(x2, e2)


@jax.jit
def kernel(x, emb0, emb1):
    e = _sc_table(emb0.reshape(-1), emb1.reshape(-1))
    x2 = x.reshape(BATCH * ROWS_PER_BATCH, LANES)
    e2 = e.reshape(ROWS_PER_BATCH, LANES)
    out2 = _tc_add(x2, e2)
    return out2.reshape(BATCH, CHANNELS, H, W)


# speedup vs baseline: 10.5327x; 10.5327x over previous
import functools

import jax
import jax.numpy as jnp
from jax.experimental import pallas as pl
from jax.experimental.pallas import tpu as pltpu

BATCH = 64
CHANNELS = 192
H = 32
W = 32

_CB = 4


def _add_body(x_ref, e0_ref, e1_ref, o_ref, et_ref):
    @pl.when(pl.program_id(0) == 0)
    def _():
        et_ref[...] = e0_ref[...][:, None, :] + e1_ref[...][None, :, :]

    o_ref[...] = x_ref[...] + et_ref[...][None]


@jax.jit
def kernel(x, emb0, emb1):
    xt = jnp.transpose(x, (0, 2, 3, 1))
    out_t = pl.pallas_call(
        _add_body,
        grid=(BATCH // _CB,),
        in_specs=[
            pl.BlockSpec((_CB, H, W, CHANNELS), lambda i: (i, 0, 0, 0)),
            pl.BlockSpec((H, CHANNELS), lambda i: (0, 0)),
            pl.BlockSpec((W, CHANNELS), lambda i: (0, 0)),
        ],
        out_specs=pl.BlockSpec((_CB, H, W, CHANNELS), lambda i: (i, 0, 0, 0)),
        out_shape=jax.ShapeDtypeStruct((BATCH, H, W, CHANNELS), jnp.float32),
        scratch_shapes=[pltpu.VMEM((H, W, CHANNELS), jnp.float32)],
        compiler_params=pltpu.CompilerParams(
            dimension_semantics=("arbitrary",),
        ),
    )(xt, emb0, emb1)
    return jnp.transpose(out_t, (0, 3, 1, 2))


# speedup vs baseline: 10.8367x; 1.0289x over previous
import functools

import jax
import jax.numpy as jnp
from jax.experimental import pallas as pl
from jax.experimental.pallas import tpu as pltpu

BATCH = 64
CHANNELS = 192
H = 32
W = 32

_CB = 8


def _add_body(x_ref, e0_ref, e1_ref, o_ref, et_ref):
    @pl.when(pl.program_id(0) == 0)
    def _():
        et_ref[...] = e0_ref[...][:, None, :] + e1_ref[...][None, :, :]

    o_ref[...] = x_ref[...] + et_ref[...][None]


@jax.jit
def kernel(x, emb0, emb1):
    xt = jnp.transpose(x, (0, 2, 3, 1))
    out_t = pl.pallas_call(
        _add_body,
        grid=(BATCH // _CB,),
        in_specs=[
            pl.BlockSpec((_CB, H, W, CHANNELS), lambda i: (i, 0, 0, 0)),
            pl.BlockSpec((H, CHANNELS), lambda i: (0, 0)),
            pl.BlockSpec((W, CHANNELS), lambda i: (0, 0)),
        ],
        out_specs=pl.BlockSpec((_CB, H, W, CHANNELS), lambda i: (i, 0, 0, 0)),
        out_shape=jax.ShapeDtypeStruct((BATCH, H, W, CHANNELS), jnp.float32),
        scratch_shapes=[pltpu.VMEM((H, W, CHANNELS), jnp.float32)],
        compiler_params=pltpu.CompilerParams(
            dimension_semantics=("arbitrary",),
        ),
    )(xt, emb0, emb1)
    return jnp.transpose(out_t, (0, 3, 1, 2))


# speedup vs baseline: 10.8643x; 1.0026x over previous
import functools

import jax
import jax.numpy as jnp
from jax.experimental import pallas as pl
from jax.experimental.pallas import tpu as pltpu

BATCH = 64
CHANNELS = 192
H = 32
W = 32

_CB = 8


def _add_body(x_ref, e0_ref, e1_ref, o_ref, et_ref):
    @pl.when(pl.program_id(0) == 0)
    def _():
        et_ref[...] = e0_ref[...][:, None, :] + e1_ref[...][None, :, :]

    o_ref[...] = x_ref[...] + et_ref[...][None]


@jax.jit
def kernel(x, emb0, emb1):
    xt = jnp.transpose(x, (0, 2, 3, 1))
    out_t = pl.pallas_call(
        _add_body,
        grid=(BATCH // _CB,),
        in_specs=[
            pl.BlockSpec((_CB, H, W, CHANNELS), lambda i: (i, 0, 0, 0)),
            pl.BlockSpec((H, CHANNELS), lambda i: (0, 0)),
            pl.BlockSpec((W, CHANNELS), lambda i: (0, 0)),
        ],
        out_specs=pl.BlockSpec((_CB, H, W, CHANNELS), lambda i: (i, 0, 0, 0)),
        out_shape=jax.ShapeDtypeStruct((BATCH, H, W, CHANNELS), jnp.float32),
        scratch_shapes=[pltpu.VMEM((H, W, CHANNELS), jnp.float32)],
        compiler_params=pltpu.CompilerParams(
            dimension_semantics=("parallel",),
        ),
    )(xt, emb0, emb1)
    return jnp.transpose(out_t, (0, 3, 1, 2))
